# baseline (speedup 1.0000x reference)
import functools

import jax
import jax.numpy as jnp
from jax import lax
from jax.experimental import pallas as pl
from jax.experimental.pallas import tpu as pltpu
from jax.experimental.pallas import tpu_sc as plsc

VOCAB = 1000000
DIM = 64
BATCH = 16384
CTX = 20

NC = 2
NS = 16
NW = NC * NS

W_PER = BATCH // NW
C_TOTAL = BATCH * CTX
C_PER = C_TOTAL // NW
C_CHUNK = 1024
C_STEPS = C_PER // C_CHUNK


def _sgns_gather(words_hbm, ctx_hbm, w_tab, c_tab, out_w, out_c,
                 idx_w, rows_w, idx_c, rows_c, sem):
    wid = lax.axis_index("s") * NC + lax.axis_index("c")

    wbase = wid * W_PER
    pltpu.sync_copy(words_hbm.at[pl.ds(wbase, W_PER)], idx_w)
    pltpu.async_copy(w_tab.at[idx_w], rows_w, sem).wait()
    pltpu.sync_copy(rows_w, out_w.at[pl.ds(wbase, W_PER)])

    cbase = wid * C_PER

    def body(i, carry):
        b = cbase + i * C_CHUNK
        pltpu.sync_copy(ctx_hbm.at[pl.ds(b, C_CHUNK)], idx_c)
        pltpu.async_copy(c_tab.at[idx_c], rows_c, sem).wait()
        pltpu.sync_copy(rows_c, out_c.at[pl.ds(b, C_CHUNK)])
        return carry

    lax.fori_loop(0, C_STEPS, body, 0)


@jax.jit
def _run(words, contexts_flat, w_embedding, c_embedding):
    mesh = plsc.VectorSubcoreMesh(core_axis_name="c", subcore_axis_name="s")
    return pl.kernel(
        _sgns_gather,
        mesh=mesh,
        out_type=[
            jax.ShapeDtypeStruct((BATCH, DIM), jnp.float32),
            jax.ShapeDtypeStruct((C_TOTAL, DIM), jnp.float32),
        ],
        scratch_types=[
            pltpu.VMEM((W_PER,), jnp.int32),
            pltpu.VMEM((W_PER, DIM), jnp.float32),
            pltpu.VMEM((C_CHUNK,), jnp.int32),
            pltpu.VMEM((C_CHUNK, DIM), jnp.float32),
            pltpu.SemaphoreType.DMA,
        ],
        compiler_params=pltpu.CompilerParams(use_tc_tiling_on_sc=False),
    )(words, contexts_flat, w_embedding, c_embedding)


def kernel(words, contexts, w_embedding, c_embedding):
    out_w, out_c = _run(words, contexts.reshape(-1), w_embedding, c_embedding)
    return out_w, out_c.reshape(BATCH, CTX, DIM)


# speedup vs baseline: 1.0060x; 1.0060x over previous
import jax
import jax.numpy as jnp
from jax import lax
from jax.experimental import pallas as pl
from jax.experimental.pallas import tpu as pltpu
from jax.experimental.pallas import tpu_sc as plsc

VOCAB = 1000000
DIM = 64
BATCH = 16384
CTX = 20

NC = 2
NS = 16
NW = NC * NS

W_PER = BATCH // NW
C_TOTAL = BATCH * CTX
C_PER = C_TOTAL // NW
C_CHUNK = 512
C_STEPS = C_PER // C_CHUNK
NBUF = 2
C_GROUPS = C_STEPS // NBUF


def _sgns_gather(words_hbm, ctx_hbm, w_tab, c_tab, out_w, out_c,
                 idx_w, idx_c, rows_w, rows_c0, rows_c1, wsem, sem0, sem1):
    wid = lax.axis_index("s") * NC + lax.axis_index("c")
    wbase = wid * W_PER
    cbase = wid * C_PER
    rows_c = (rows_c0, rows_c1)
    sems = (sem0, sem1)

    pltpu.sync_copy(words_hbm.at[pl.ds(wbase, W_PER)], idx_w)
    pltpu.sync_copy(ctx_hbm.at[pl.ds(cbase, C_PER)], idx_c)

    w_copy = pltpu.async_copy(w_tab.at[idx_w], rows_w, wsem)

    def c_gather_start(g, b):
        return pltpu.async_copy(
            c_tab.at[idx_c.at[pl.ds(g * C_CHUNK, C_CHUNK)]], rows_c[b],
            sems[b])

    for b in range(NBUF):
        c_gather_start(b, b)

    def group(g0, carry):
        for b in range(NBUF):
            g = g0 * NBUF + b
            pltpu.make_async_copy(
                c_tab.at[idx_c.at[pl.ds(g * C_CHUNK, C_CHUNK)]], rows_c[b],
                sems[b]).wait()
            pltpu.sync_copy(rows_c[b],
                            out_c.at[pl.ds(cbase + g * C_CHUNK, C_CHUNK)])
            c_gather_start(g + NBUF, b)
        return carry

    lax.fori_loop(0, C_GROUPS - 1, group, 0)

    for b in range(NBUF):
        g = (C_GROUPS - 1) * NBUF + b
        pltpu.make_async_copy(
            c_tab.at[idx_c.at[pl.ds(g * C_CHUNK, C_CHUNK)]], rows_c[b],
            sems[b]).wait()
        pltpu.sync_copy(rows_c[b],
                        out_c.at[pl.ds(cbase + g * C_CHUNK, C_CHUNK)])

    w_copy.wait()
    pltpu.sync_copy(rows_w, out_w.at[pl.ds(wbase, W_PER)])


@jax.jit
def _run(words, contexts_flat, w_embedding, c_embedding):
    mesh = plsc.VectorSubcoreMesh(core_axis_name="c", subcore_axis_name="s")
    return pl.kernel(
        _sgns_gather,
        mesh=mesh,
        out_type=[
            jax.ShapeDtypeStruct((BATCH, DIM), jnp.float32),
            jax.ShapeDtypeStruct((C_TOTAL, DIM), jnp.float32),
        ],
        scratch_types=[
            pltpu.VMEM((W_PER,), jnp.int32),
            pltpu.VMEM((C_PER,), jnp.int32),
            pltpu.VMEM((W_PER, DIM), jnp.float32),
            pltpu.VMEM((C_CHUNK, DIM), jnp.float32),
            pltpu.VMEM((C_CHUNK, DIM), jnp.float32),
            pltpu.SemaphoreType.DMA,
            pltpu.SemaphoreType.DMA,
            pltpu.SemaphoreType.DMA,
        ],
        compiler_params=pltpu.CompilerParams(use_tc_tiling_on_sc=False),
    )(words, contexts_flat, w_embedding, c_embedding)


def kernel(words, contexts, w_embedding, c_embedding):
    out_w, out_c = _run(words, contexts.reshape(-1), w_embedding, c_embedding)
    return out_w, out_c.reshape(BATCH, CTX, DIM)
